# baseline (speedup 1.0000x reference)
import functools

import jax
import jax.numpy as jnp
from jax import lax
from jax.experimental import pallas as pl
from jax.experimental.pallas import tpu as pltpu
from jax.experimental.pallas import tpu_sc as plsc

D = 128
W = 144
NC = 2
NS = 16
CH = 128
SPLIT0 = 0.70


def _row_block(n):
    for b in (1000, 500, 250, 200, 125, 100, 50, 40, 25, 20, 10, 8, 5, 4, 2, 1):
        if n % b == 0:
            return b
    return n


def _node_math(t, nr_raw):
    nr = nr_raw / jnp.maximum(jnp.sqrt(jnp.sum(nr_raw * nr_raw)), 1e-12)
    p = jnp.sum(t * nr, axis=1, keepdims=True)
    q = jnp.sqrt(jnp.sum(t * t, axis=1, keepdims=True))
    w = jnp.exp(-p / jnp.maximum(q, 1e-12))
    lane = lax.broadcasted_iota(jnp.int32, (t.shape[0], W - D), 1)
    tail = jnp.where(lane == 0, w, jnp.zeros_like(w))
    return jnp.concatenate([t * w, tail], axis=1)


def _prep_body(x_ref, nr_ref, f_ref, h_ref):
    t = jnp.tanh(x_ref[...])
    f_ref[...] = t
    h_ref[...] = _node_math(t, nr_ref[...])


def _finish_body(p0_ref, p1_ref, nr_ref, f_ref, h_ref):
    s = p0_ref[...] + p1_ref[...]
    t = jnp.tanh(s[:, :D] / jnp.maximum(s[:, D:D + 1], 1e-12))
    f_ref[...] = t
    h_ref[...] = _node_math(t, nr_ref[...])


def _dense_call(body, n, ins, widths):
    b = _row_block(n)
    grid = (n // b,)
    in_specs = []
    for a in ins:
        if a.shape[0] == 1:
            in_specs.append(pl.BlockSpec(a.shape, lambda i: (0, 0)))
        else:
            in_specs.append(pl.BlockSpec((b, a.shape[1]), lambda i: (i, 0)))
    out_specs = [pl.BlockSpec((b, w), lambda i: (i, 0)) for w in widths]
    out_shape = [jax.ShapeDtypeStruct((n, w), jnp.float32) for w in widths]
    return pl.pallas_call(
        body, grid=grid, in_specs=in_specs, out_specs=out_specs,
        out_shape=out_shape)(*ins)


def _make_spmm(n_pad, e_pad, k0):
    tot = e_pad // (NS * CH)
    k1 = tot - k0
    rpt = n_pad // NS
    nfull = rpt // CH
    rem = rpt - nfull * CH
    mesh = plsc.VectorSubcoreMesh(core_axis_name="c", subcore_axis_name="s")

    @functools.partial(
        pl.kernel,
        out_type=jax.ShapeDtypeStruct((NC * n_pad, W), jnp.float32),
        mesh=mesh,
        compiler_params=pltpu.CompilerParams(use_tc_tiling_on_sc=False),
        scratch_types=[
            pltpu.VMEM((3, CH), jnp.int32),
            pltpu.VMEM((3, CH), jnp.int32),
            pltpu.VMEM((2 * CH, W), jnp.float32),
            pltpu.VMEM_SHARED((n_pad, W), jnp.float32),
            pltpu.SemaphoreType.DMA,
            pltpu.SemaphoreType.DMA,
            pltpu.SemaphoreType.DMA,
        ],
    )
    def spmm(h_hbm, col_hbm, row_hbm, out_hbm, colv, rowv, buf, acc,
             isem, gsem, ssem):
        c = lax.axis_index("c")
        s = lax.axis_index("s")
        myk = jnp.where(c == 0, k0, k1)

        zoffs = list(range(0, W - 15, 16))
        if W % 16:
            zoffs.append(W - 16)

        def zrow(i, carry):
            for g in zoffs:
                buf[i, pl.ds(g, 16)] = jnp.zeros((16,), jnp.float32)
            return carry
        lax.fori_loop(0, CH, zrow, 0)

        r0 = s * rpt

        def zcopy(j, carry):
            pltpu.sync_copy(buf.at[pl.ds(0, CH)],
                            acc.at[pl.ds(r0 + j * CH, CH)])
            return carry
        lax.fori_loop(0, nfull, zcopy, 0)
        if rem:
            pltpu.sync_copy(buf.at[pl.ds(0, rem)],
                            acc.at[pl.ds(r0 + nfull * CH, rem)])

        cbase = jnp.where(c == 0, s * k0, NS * k0 + s * k1)

        def start_idx(j):
            sl = lax.rem(j, 3)
            pltpu.async_copy(col_hbm.at[cbase + j], colv.at[sl], isem)
            pltpu.async_copy(row_hbm.at[cbase + j], rowv.at[sl], isem)

        def wait_idx():
            for _ in range(2):
                pltpu.make_async_copy(col_hbm.at[0], colv.at[0], isem).wait()

        def bufsl(j):
            return buf.at[pl.ds(lax.rem(j, 2) * CH, CH)]

        def start_gather(j):
            pltpu.async_copy(h_hbm.at[colv.at[lax.rem(j, 3)]], bufsl(j),
                             gsem)

        def wait_gather(j):
            pltpu.make_async_copy(
                h_hbm.at[colv.at[lax.rem(j, 3)]], bufsl(j), gsem).wait()

        def wait_scatter(j):
            pltpu.make_async_copy(
                bufsl(j), acc.at[rowv.at[lax.rem(j, 3)]], ssem).wait()

        start_idx(0)
        start_idx(1)
        wait_idx()
        start_gather(0)
        plsc.subcore_barrier()

        def echunk(j, carry):
            @pl.when(j >= 1)
            def _():
                wait_scatter(j - 1)

            wait_gather(j)

            @pl.when(j + 1 < myk)
            def _():
                wait_idx()
                start_gather(j + 1)

            @pl.when(j + 2 < myk)
            def _():
                start_idx(j + 2)

            pltpu.async_copy(bufsl(j), acc.at[rowv.at[lax.rem(j, 3)]],
                             ssem, add=True)
            return carry
        lax.fori_loop(0, myk, echunk, 0)
        wait_scatter(myk - 1)
        plsc.subcore_barrier()

        pltpu.sync_copy(acc.at[pl.ds(r0, rpt)],
                        out_hbm.at[pl.ds(c * n_pad + r0, rpt)])

    return spmm


def kernel(features, rel_emb, adj_index, none_relation):
    del rel_emb
    n, d = features.shape
    assert d == D and n % NS == 0
    e = adj_index.shape[0]
    grp = NS * CH
    e_pad = ((e + grp - 1) // grp) * grp
    n_pad = ((n + NS * 8 - 1) // (NS * 8)) * (NS * 8)
    k0 = round(e_pad // grp * SPLIT0)

    adj = adj_index.astype(jnp.int32)
    colp = jnp.concatenate(
        [adj[:, 1], jnp.full((e_pad - e,), n, jnp.int32)]).reshape(-1, CH)
    rowp = jnp.concatenate(
        [adj[:, 0], jnp.zeros((e_pad - e,), jnp.int32)]).reshape(-1, CH)
    nr2 = none_relation.reshape(1, D).astype(jnp.float32)
    zpad = jnp.zeros((8, W), jnp.float32)

    spmm = _make_spmm(n_pad, e_pad, k0)

    f0, h0 = _dense_call(_prep_body, n, [features, nr2], [D, W])
    p = spmm(jnp.concatenate([h0, zpad]), colp, rowp)
    f1, h1 = _dense_call(
        _finish_body, n, [p[:n], p[n_pad:n_pad + n], nr2], [D, W])
    p2 = spmm(jnp.concatenate([h1, zpad]), colp, rowp)
    f2, _ = _dense_call(
        _finish_body, n, [p2[:n], p2[n_pad:n_pad + n], nr2], [D, W])
    return jnp.concatenate([f0, f1, f2], axis=1)


# speedup vs baseline: 1.0341x; 1.0341x over previous
import functools

import jax
import jax.numpy as jnp
from jax import lax
from jax.experimental import pallas as pl
from jax.experimental.pallas import tpu as pltpu
from jax.experimental.pallas import tpu_sc as plsc

D = 128
W = 144
NC = 2
NS = 16
CH = 128
SPLIT0 = 0.60


def _row_block(n):
    for b in (1000, 500, 250, 200, 125, 100, 50, 40, 25, 20, 10, 8, 5, 4, 2, 1):
        if n % b == 0:
            return b
    return n


def _node_math(t, nr_raw):
    nr = nr_raw / jnp.maximum(jnp.sqrt(jnp.sum(nr_raw * nr_raw)), 1e-12)
    p = jnp.sum(t * nr, axis=1, keepdims=True)
    q = jnp.sqrt(jnp.sum(t * t, axis=1, keepdims=True))
    w = jnp.exp(-p / jnp.maximum(q, 1e-12))
    lane = lax.broadcasted_iota(jnp.int32, (t.shape[0], W - D), 1)
    tail = jnp.where(lane == 0, w, jnp.zeros_like(w))
    return jnp.concatenate([t * w, tail], axis=1)


def _prep_body(x_ref, nr_ref, f_ref, h_ref):
    t = jnp.tanh(x_ref[...])
    f_ref[...] = t
    h_ref[...] = _node_math(t, nr_ref[...])


def _finish_body(p0_ref, p1_ref, nr_ref, f_ref, h_ref):
    s = p0_ref[...] + p1_ref[...]
    t = jnp.tanh(s[:, :D] / jnp.maximum(s[:, D:D + 1], 1e-12))
    f_ref[...] = t
    h_ref[...] = _node_math(t, nr_ref[...])


def _dense_call(body, n, ins, widths):
    b = _row_block(n)
    grid = (n // b,)
    in_specs = []
    for a in ins:
        if a.shape[0] == 1:
            in_specs.append(pl.BlockSpec(a.shape, lambda i: (0, 0)))
        else:
            in_specs.append(pl.BlockSpec((b, a.shape[1]), lambda i: (i, 0)))
    out_specs = [pl.BlockSpec((b, w), lambda i: (i, 0)) for w in widths]
    out_shape = [jax.ShapeDtypeStruct((n, w), jnp.float32) for w in widths]
    return pl.pallas_call(
        body, grid=grid, in_specs=in_specs, out_specs=out_specs,
        out_shape=out_shape)(*ins)


def _make_spmm(n_pad, e_pad, k0):
    tot = e_pad // (NS * CH)
    k1 = tot - k0
    rpt = n_pad // NS
    nfull = rpt // CH
    rem = rpt - nfull * CH
    mesh = plsc.VectorSubcoreMesh(core_axis_name="c", subcore_axis_name="s")

    @functools.partial(
        pl.kernel,
        out_type=jax.ShapeDtypeStruct((NC * n_pad, W), jnp.float32),
        mesh=mesh,
        compiler_params=pltpu.CompilerParams(use_tc_tiling_on_sc=False),
        scratch_types=[
            pltpu.VMEM((3, CH), jnp.int32),
            pltpu.VMEM((3, CH), jnp.int32),
            pltpu.VMEM((2 * CH, W), jnp.float32),
            pltpu.VMEM_SHARED((n_pad, W), jnp.float32),
            pltpu.SemaphoreType.DMA,
            pltpu.SemaphoreType.DMA,
            pltpu.SemaphoreType.DMA,
        ],
    )
    def spmm(h_hbm, col_hbm, row_hbm, out_hbm, colv, rowv, buf, acc,
             isem, gsem, ssem):
        c = lax.axis_index("c")
        s = lax.axis_index("s")
        myk = jnp.where(c == 0, k0, k1)

        zoffs = list(range(0, W - 15, 16))
        if W % 16:
            zoffs.append(W - 16)

        def zrow(i, carry):
            for g in zoffs:
                buf[i, pl.ds(g, 16)] = jnp.zeros((16,), jnp.float32)
            return carry
        lax.fori_loop(0, CH, zrow, 0)

        r0 = s * rpt

        def zcopy(j, carry):
            pltpu.sync_copy(buf.at[pl.ds(0, CH)],
                            acc.at[pl.ds(r0 + j * CH, CH)])
            return carry
        lax.fori_loop(0, nfull, zcopy, 0)
        if rem:
            pltpu.sync_copy(buf.at[pl.ds(0, rem)],
                            acc.at[pl.ds(r0 + nfull * CH, rem)])

        cbase = jnp.where(c == 0, s * k0, NS * k0 + s * k1)

        def start_idx(j):
            sl = lax.rem(j, 3)
            pltpu.async_copy(col_hbm.at[cbase + j], colv.at[sl], isem)
            pltpu.async_copy(row_hbm.at[cbase + j], rowv.at[sl], isem)

        def wait_idx():
            for _ in range(2):
                pltpu.make_async_copy(col_hbm.at[0], colv.at[0], isem).wait()

        def bufsl(j):
            return buf.at[pl.ds(lax.rem(j, 2) * CH, CH)]

        def start_gather(j):
            pltpu.async_copy(h_hbm.at[colv.at[lax.rem(j, 3)]], bufsl(j),
                             gsem)

        def wait_gather(j):
            pltpu.make_async_copy(
                h_hbm.at[colv.at[lax.rem(j, 3)]], bufsl(j), gsem).wait()

        def wait_scatter(j):
            pltpu.make_async_copy(
                bufsl(j), acc.at[rowv.at[lax.rem(j, 3)]], ssem).wait()

        start_idx(0)
        start_idx(1)
        wait_idx()
        start_gather(0)
        plsc.subcore_barrier()

        def echunk(j, carry):
            @pl.when(j >= 1)
            def _():
                wait_scatter(j - 1)

            wait_gather(j)

            @pl.when(j + 1 < myk)
            def _():
                wait_idx()
                start_gather(j + 1)

            @pl.when(j + 2 < myk)
            def _():
                start_idx(j + 2)

            pltpu.async_copy(bufsl(j), acc.at[rowv.at[lax.rem(j, 3)]],
                             ssem, add=True)
            return carry
        lax.fori_loop(0, myk, echunk, 0)
        wait_scatter(myk - 1)
        plsc.subcore_barrier()

        pltpu.sync_copy(acc.at[pl.ds(r0, rpt)],
                        out_hbm.at[pl.ds(c * n_pad + r0, rpt)])

    return spmm


def kernel(features, rel_emb, adj_index, none_relation):
    del rel_emb
    n, d = features.shape
    assert d == D and n % NS == 0
    e = adj_index.shape[0]
    grp = NS * CH
    e_pad = ((e + grp - 1) // grp) * grp
    n_pad = ((n + NS * 8 - 1) // (NS * 8)) * (NS * 8)
    k0 = round(e_pad // grp * SPLIT0)

    adj = adj_index.astype(jnp.int32)
    colp = jnp.concatenate(
        [adj[:, 1], jnp.full((e_pad - e,), n, jnp.int32)]).reshape(-1, CH)
    rowp = jnp.concatenate(
        [adj[:, 0], jnp.zeros((e_pad - e,), jnp.int32)]).reshape(-1, CH)
    nr2 = none_relation.reshape(1, D).astype(jnp.float32)
    zpad = jnp.zeros((8, W), jnp.float32)

    spmm = _make_spmm(n_pad, e_pad, k0)

    f0, h0 = _dense_call(_prep_body, n, [features, nr2], [D, W])
    p = spmm(jnp.concatenate([h0, zpad]), colp, rowp)
    f1, h1 = _dense_call(
        _finish_body, n, [p[:n], p[n_pad:n_pad + n], nr2], [D, W])
    p2 = spmm(jnp.concatenate([h1, zpad]), colp, rowp)
    f2, _ = _dense_call(
        _finish_body, n, [p2[:n], p2[n_pad:n_pad + n], nr2], [D, W])
    return jnp.concatenate([f0, f1, f2], axis=1)


# speedup vs baseline: 1.0441x; 1.0096x over previous
import functools

import jax
import jax.numpy as jnp
from jax import lax
from jax.experimental import pallas as pl
from jax.experimental.pallas import tpu as pltpu
from jax.experimental.pallas import tpu_sc as plsc

D = 128
W = 144
NC = 2
NS = 16
CH = 128
SPLIT0 = 0.63


def _row_block(n):
    for b in (1000, 500, 250, 200, 125, 100, 50, 40, 25, 20, 10, 8, 5, 4, 2, 1):
        if n % b == 0:
            return b
    return n


def _node_math(t, nr_raw):
    nr = nr_raw / jnp.maximum(jnp.sqrt(jnp.sum(nr_raw * nr_raw)), 1e-12)
    p = jnp.sum(t * nr, axis=1, keepdims=True)
    q = jnp.sqrt(jnp.sum(t * t, axis=1, keepdims=True))
    w = jnp.exp(-p / jnp.maximum(q, 1e-12))
    lane = lax.broadcasted_iota(jnp.int32, (t.shape[0], W - D), 1)
    tail = jnp.where(lane == 0, w, jnp.zeros_like(w))
    return jnp.concatenate([t * w, tail], axis=1)


def _prep_body(x_ref, nr_ref, f_ref, h_ref):
    t = jnp.tanh(x_ref[...])
    f_ref[...] = t
    h_ref[...] = _node_math(t, nr_ref[...])


def _finish_body(p0_ref, p1_ref, nr_ref, f_ref, h_ref):
    s = p0_ref[...] + p1_ref[...]
    t = jnp.tanh(s[:, :D] / jnp.maximum(s[:, D:D + 1], 1e-12))
    f_ref[...] = t
    h_ref[...] = _node_math(t, nr_ref[...])


def _dense_call(body, n, ins, widths):
    b = _row_block(n)
    grid = (n // b,)
    in_specs = []
    for a in ins:
        if a.shape[0] == 1:
            in_specs.append(pl.BlockSpec(a.shape, lambda i: (0, 0)))
        else:
            in_specs.append(pl.BlockSpec((b, a.shape[1]), lambda i: (i, 0)))
    out_specs = [pl.BlockSpec((b, w), lambda i: (i, 0)) for w in widths]
    out_shape = [jax.ShapeDtypeStruct((n, w), jnp.float32) for w in widths]
    return pl.pallas_call(
        body, grid=grid, in_specs=in_specs, out_specs=out_specs,
        out_shape=out_shape)(*ins)


def _make_spmm(n_pad, e_pad, k0):
    tot = e_pad // (NS * CH)
    k1 = tot - k0
    rpt = n_pad // NS
    nfull = rpt // CH
    rem = rpt - nfull * CH
    mesh = plsc.VectorSubcoreMesh(core_axis_name="c", subcore_axis_name="s")

    @functools.partial(
        pl.kernel,
        out_type=jax.ShapeDtypeStruct((NC * n_pad, W), jnp.float32),
        mesh=mesh,
        compiler_params=pltpu.CompilerParams(use_tc_tiling_on_sc=False),
        scratch_types=[
            pltpu.VMEM((3, CH), jnp.int32),
            pltpu.VMEM((3, CH), jnp.int32),
            pltpu.VMEM((2 * CH, W), jnp.float32),
            pltpu.VMEM_SHARED((n_pad, W), jnp.float32),
            pltpu.SemaphoreType.DMA,
            pltpu.SemaphoreType.DMA,
            pltpu.SemaphoreType.DMA,
        ],
    )
    def spmm(h_hbm, col_hbm, row_hbm, out_hbm, colv, rowv, buf, acc,
             isem, gsem, ssem):
        c = lax.axis_index("c")
        s = lax.axis_index("s")
        myk = jnp.where(c == 0, k0, k1)

        zoffs = list(range(0, W - 15, 16))
        if W % 16:
            zoffs.append(W - 16)

        def zrow(i, carry):
            for g in zoffs:
                buf[i, pl.ds(g, 16)] = jnp.zeros((16,), jnp.float32)
            return carry
        lax.fori_loop(0, CH, zrow, 0)

        r0 = s * rpt

        def zcopy(j, carry):
            pltpu.sync_copy(buf.at[pl.ds(0, CH)],
                            acc.at[pl.ds(r0 + j * CH, CH)])
            return carry
        lax.fori_loop(0, nfull, zcopy, 0)
        if rem:
            pltpu.sync_copy(buf.at[pl.ds(0, rem)],
                            acc.at[pl.ds(r0 + nfull * CH, rem)])

        cbase = jnp.where(c == 0, s * k0, NS * k0 + s * k1)

        def start_idx(j):
            sl = lax.rem(j, 3)
            pltpu.async_copy(col_hbm.at[cbase + j], colv.at[sl], isem)
            pltpu.async_copy(row_hbm.at[cbase + j], rowv.at[sl], isem)

        def wait_idx():
            for _ in range(2):
                pltpu.make_async_copy(col_hbm.at[0], colv.at[0], isem).wait()

        def bufsl(j):
            return buf.at[pl.ds(lax.rem(j, 2) * CH, CH)]

        def start_gather(j):
            pltpu.async_copy(h_hbm.at[colv.at[lax.rem(j, 3)]], bufsl(j),
                             gsem)

        def wait_gather(j):
            pltpu.make_async_copy(
                h_hbm.at[colv.at[lax.rem(j, 3)]], bufsl(j), gsem).wait()

        def wait_scatter(j):
            pltpu.make_async_copy(
                bufsl(j), acc.at[rowv.at[lax.rem(j, 3)]], ssem).wait()

        start_idx(0)
        start_idx(1)
        wait_idx()
        start_gather(0)
        plsc.subcore_barrier()

        def echunk(j, carry):
            @pl.when(j >= 1)
            def _():
                wait_scatter(j - 1)

            wait_gather(j)

            @pl.when(j + 1 < myk)
            def _():
                wait_idx()
                start_gather(j + 1)

            @pl.when(j + 2 < myk)
            def _():
                start_idx(j + 2)

            pltpu.async_copy(bufsl(j), acc.at[rowv.at[lax.rem(j, 3)]],
                             ssem, add=True)
            return carry
        lax.fori_loop(0, myk, echunk, 0)
        wait_scatter(myk - 1)
        plsc.subcore_barrier()

        pltpu.sync_copy(acc.at[pl.ds(r0, rpt)],
                        out_hbm.at[pl.ds(c * n_pad + r0, rpt)])

    return spmm


def kernel(features, rel_emb, adj_index, none_relation):
    del rel_emb
    n, d = features.shape
    assert d == D and n % NS == 0
    e = adj_index.shape[0]
    grp = NS * CH
    e_pad = ((e + grp - 1) // grp) * grp
    n_pad = ((n + NS * 8 - 1) // (NS * 8)) * (NS * 8)
    k0 = round(e_pad // grp * SPLIT0)

    adj = adj_index.astype(jnp.int32)
    colp = jnp.concatenate(
        [adj[:, 1], jnp.full((e_pad - e,), n, jnp.int32)]).reshape(-1, CH)
    rowp = jnp.concatenate(
        [adj[:, 0], jnp.zeros((e_pad - e,), jnp.int32)]).reshape(-1, CH)
    nr2 = none_relation.reshape(1, D).astype(jnp.float32)
    zpad = jnp.zeros((8, W), jnp.float32)

    spmm = _make_spmm(n_pad, e_pad, k0)

    f0, h0 = _dense_call(_prep_body, n, [features, nr2], [D, W])
    p = spmm(jnp.concatenate([h0, zpad]), colp, rowp)
    f1, h1 = _dense_call(
        _finish_body, n, [p[:n], p[n_pad:n_pad + n], nr2], [D, W])
    p2 = spmm(jnp.concatenate([h1, zpad]), colp, rowp)
    f2, _ = _dense_call(
        _finish_body, n, [p2[:n], p2[n_pad:n_pad + n], nr2], [D, W])
    return jnp.concatenate([f0, f1, f2], axis=1)


# speedup vs baseline: 1.1172x; 1.0700x over previous
import functools

import jax
import jax.numpy as jnp
from jax import lax
from jax.experimental import pallas as pl
from jax.experimental.pallas import tpu as pltpu
from jax.experimental.pallas import tpu_sc as plsc

D = 128
W = 144
NC = 2
NS = 16
CH = 128
SPLIT0 = 0.63


def _row_block(n):
    for b in range(1200, 7, -8):
        if n % b == 0:
            return b
    return n


def _node_math(t, nr_raw, valid):
    nr = nr_raw / jnp.maximum(jnp.sqrt(jnp.sum(nr_raw * nr_raw)), 1e-12)
    p = jnp.sum(t * nr, axis=1, keepdims=True)
    q = jnp.sqrt(jnp.sum(t * t, axis=1, keepdims=True))
    w = jnp.exp(-p / jnp.maximum(q, 1e-12))
    lane = lax.broadcasted_iota(jnp.int32, (t.shape[0], W - D), 1)
    tail = jnp.where(lane == 0, w, jnp.zeros_like(w))
    h = jnp.concatenate([t * w, tail], axis=1)
    return jnp.where(valid, h, jnp.zeros_like(h))


def _valid_rows(n, b, rows):
    i = pl.program_id(0)
    rowid = i * b + lax.broadcasted_iota(jnp.int32, (rows, 1), 0)
    return rowid < n


def _make_prep_body(n, b):
    def body(x_ref, nr_ref, f_ref, h_ref):
        valid = _valid_rows(n, b, x_ref.shape[0])
        t = jnp.tanh(jnp.where(valid, x_ref[...], 0.0))
        f_ref[...] = t
        h_ref[...] = _node_math(t, nr_ref[...], valid)
    return body


def _make_finish_body(n, b):
    def body(p0_ref, p1_ref, nr_ref, f_ref, h_ref):
        valid = _valid_rows(n, b, p0_ref.shape[1])
        s = p0_ref[0] + p1_ref[0]
        t = jnp.tanh(s[:, :D] / jnp.maximum(s[:, D:D + 1], 1e-12))
        t = jnp.where(valid, t, 0.0)
        f_ref[...] = t
        h_ref[...] = _node_math(t, nr_ref[...], valid)
    return body


def _dense_call(body, n, n_h, ins):
    b = _row_block(n_h)
    grid = (n_h // b,)
    in_specs = []
    for a, spec in ins:
        in_specs.append(spec(b))
    out_specs = [pl.BlockSpec((b, D), lambda i: (i, 0)),
                 pl.BlockSpec((b, W), lambda i: (i, 0))]
    out_shape = [jax.ShapeDtypeStruct((n, D), jnp.float32),
                 jax.ShapeDtypeStruct((n_h, W), jnp.float32)]
    return pl.pallas_call(
        body, grid=grid, in_specs=in_specs, out_specs=out_specs,
        out_shape=out_shape)(*[a for a, _ in ins])


def _make_spmm(n_pad, e_pad, k0):
    tot = e_pad // (NS * CH)
    k1 = tot - k0
    rpt = n_pad // NS
    nfull = rpt // CH
    rem = rpt - nfull * CH
    mesh = plsc.VectorSubcoreMesh(core_axis_name="c", subcore_axis_name="s")

    @functools.partial(
        pl.kernel,
        out_type=jax.ShapeDtypeStruct((NC * n_pad, W), jnp.float32),
        mesh=mesh,
        compiler_params=pltpu.CompilerParams(use_tc_tiling_on_sc=False),
        scratch_types=[
            pltpu.VMEM((3, CH), jnp.int32),
            pltpu.VMEM((3, CH), jnp.int32),
            pltpu.VMEM((2 * CH, W), jnp.float32),
            pltpu.VMEM_SHARED((n_pad, W), jnp.float32),
            pltpu.SemaphoreType.DMA,
            pltpu.SemaphoreType.DMA,
            pltpu.SemaphoreType.DMA,
        ],
    )
    def spmm(h_hbm, col_hbm, row_hbm, out_hbm, colv, rowv, buf, acc,
             isem, gsem, ssem):
        c = lax.axis_index("c")
        s = lax.axis_index("s")
        myk = jnp.where(c == 0, k0, k1)

        zoffs = list(range(0, W - 15, 16))
        if W % 16:
            zoffs.append(W - 16)

        def zrow(i, carry):
            for g in zoffs:
                buf[i, pl.ds(g, 16)] = jnp.zeros((16,), jnp.float32)
            return carry
        lax.fori_loop(0, CH, zrow, 0)

        r0 = s * rpt

        def zcopy(j, carry):
            pltpu.sync_copy(buf.at[pl.ds(0, CH)],
                            acc.at[pl.ds(r0 + j * CH, CH)])
            return carry
        lax.fori_loop(0, nfull, zcopy, 0)
        if rem:
            pltpu.sync_copy(buf.at[pl.ds(0, rem)],
                            acc.at[pl.ds(r0 + nfull * CH, rem)])

        cbase = jnp.where(c == 0, s * k0, NS * k0 + s * k1)

        def start_idx(j):
            sl = lax.rem(j, 3)
            pltpu.async_copy(col_hbm.at[cbase + j], colv.at[sl], isem)
            pltpu.async_copy(row_hbm.at[cbase + j], rowv.at[sl], isem)

        def wait_idx():
            for _ in range(2):
                pltpu.make_async_copy(col_hbm.at[0], colv.at[0], isem).wait()

        def bufsl(j):
            return buf.at[pl.ds(lax.rem(j, 2) * CH, CH)]

        def start_gather(j):
            pltpu.async_copy(h_hbm.at[colv.at[lax.rem(j, 3)]], bufsl(j),
                             gsem)

        def wait_gather(j):
            pltpu.make_async_copy(
                h_hbm.at[colv.at[lax.rem(j, 3)]], bufsl(j), gsem).wait()

        def wait_scatter(j):
            pltpu.make_async_copy(
                bufsl(j), acc.at[rowv.at[lax.rem(j, 3)]], ssem).wait()

        start_idx(0)
        start_idx(1)
        wait_idx()
        start_gather(0)
        plsc.subcore_barrier()

        def echunk(j, carry):
            @pl.when(j >= 1)
            def _():
                wait_scatter(j - 1)

            wait_gather(j)

            @pl.when(j + 1 < myk)
            def _():
                wait_idx()
                start_gather(j + 1)

            @pl.when(j + 2 < myk)
            def _():
                start_idx(j + 2)

            pltpu.async_copy(bufsl(j), acc.at[rowv.at[lax.rem(j, 3)]],
                             ssem, add=True)
            return carry
        lax.fori_loop(0, myk, echunk, 0)
        wait_scatter(myk - 1)
        plsc.subcore_barrier()

        pltpu.sync_copy(acc.at[pl.ds(r0, rpt)],
                        out_hbm.at[pl.ds(c * n_pad + r0, rpt)])

    return spmm


def kernel(features, rel_emb, adj_index, none_relation):
    del rel_emb
    n, d = features.shape
    assert d == D and n % NS == 0
    e = adj_index.shape[0]
    grp = NS * CH
    e_pad = ((e + grp - 1) // grp) * grp
    n_pad = ((n + NS * 8 - 1) // (NS * 8)) * (NS * 8)
    k0 = round(e_pad // grp * SPLIT0)

    adj = adj_index.astype(jnp.int32)
    colp = jnp.concatenate(
        [adj[:, 1], jnp.full((e_pad - e,), n, jnp.int32)]).reshape(-1, CH)
    rowp = jnp.concatenate(
        [adj[:, 0], jnp.zeros((e_pad - e,), jnp.int32)]).reshape(-1, CH)
    nr2 = none_relation.reshape(1, D).astype(jnp.float32)
    n_h = n + 8

    spmm = _make_spmm(n_pad, e_pad, k0)

    def full_spec(a):
        return lambda b: pl.BlockSpec(a.shape, lambda i: (0, 0))

    def row_spec(width):
        return lambda b: pl.BlockSpec((b, width), lambda i: (i, 0))

    def part_spec(core):
        return lambda b: pl.BlockSpec((1, b, W), lambda i: (core, i, 0))

    prep = _make_prep_body(n, _row_block(n_h))
    finish = _make_finish_body(n, _row_block(n_h))

    f0, h0 = _dense_call(
        prep, n, n_h, [(features, row_spec(D)), (nr2, full_spec(nr2))])
    p = spmm(h0, colp, rowp).reshape(NC, n_pad, W)
    f1, h1 = _dense_call(
        finish, n, n_h,
        [(p, part_spec(0)), (p, part_spec(1)), (nr2, full_spec(nr2))])
    p2 = spmm(h1, colp, rowp).reshape(NC, n_pad, W)
    f2, _ = _dense_call(
        finish, n, n_h,
        [(p2, part_spec(0)), (p2, part_spec(1)), (nr2, full_spec(nr2))])
    return jnp.concatenate([f0, f1, f2], axis=1)


# speedup vs baseline: 1.4532x; 1.3008x over previous
import functools

import jax
import jax.numpy as jnp
from jax import lax
from jax.experimental import pallas as pl
from jax.experimental.pallas import tpu as pltpu
from jax.experimental.pallas import tpu_sc as plsc

D = 128
W = 144
NC = 2
NS = 16
CH = 80
SPLIT0 = 0.63


def _row_block(n):
    for b in range(1200, 7, -8):
        if n % b == 0:
            return b
    return n


def _node_math(t, nr_raw, valid):
    nr = nr_raw / jnp.maximum(jnp.sqrt(jnp.sum(nr_raw * nr_raw)), 1e-12)
    p = jnp.sum(t * nr, axis=1, keepdims=True)
    q = jnp.sqrt(jnp.sum(t * t, axis=1, keepdims=True))
    w = jnp.exp(-p / jnp.maximum(q, 1e-12))
    lane = lax.broadcasted_iota(jnp.int32, (t.shape[0], W - D), 1)
    tail = jnp.where(lane == 0, w, jnp.zeros_like(w))
    h = jnp.concatenate([t * w, tail], axis=1)
    return jnp.where(valid, h, jnp.zeros_like(h))


def _valid_rows(n, b, rows):
    i = pl.program_id(0)
    rowid = i * b + lax.broadcasted_iota(jnp.int32, (rows, 1), 0)
    return rowid < n


def _make_prep_body(n, b):
    def body(x_ref, nr_ref, f_ref, h_ref):
        valid = _valid_rows(n, b, x_ref.shape[0])
        t = jnp.tanh(jnp.where(valid, x_ref[...], 0.0))
        f_ref[...] = t
        h_ref[...] = _node_math(t, nr_ref[...], valid)
    return body


def _make_finish_body(n, b):
    def body(p0_ref, p1_ref, nr_ref, f_ref, h_ref):
        valid = _valid_rows(n, b, p0_ref.shape[1])
        s = p0_ref[0] + p1_ref[0]
        t = jnp.tanh(s[:, :D] / jnp.maximum(s[:, D:D + 1], 1e-12))
        t = jnp.where(valid, t, 0.0)
        f_ref[...] = t
        h_ref[...] = _node_math(t, nr_ref[...], valid)
    return body


def _dense_call(body, n, n_h, ins):
    b = _row_block(n_h)
    grid = (n_h // b,)
    in_specs = []
    for a, spec in ins:
        in_specs.append(spec(b))
    out_specs = [pl.BlockSpec((b, D), lambda i: (i, 0)),
                 pl.BlockSpec((b, W), lambda i: (i, 0))]
    out_shape = [jax.ShapeDtypeStruct((n, D), jnp.float32),
                 jax.ShapeDtypeStruct((n_h, W), jnp.float32)]
    return pl.pallas_call(
        body, grid=grid, in_specs=in_specs, out_specs=out_specs,
        out_shape=out_shape)(*[a for a, _ in ins])


def _make_spmm(n_pad, e_pad, k0):
    tot = e_pad // (NS * CH)
    k1 = tot - k0
    rpt = n_pad // NS
    nfull = rpt // CH
    rem = rpt - nfull * CH
    mesh = plsc.VectorSubcoreMesh(core_axis_name="c", subcore_axis_name="s")

    @functools.partial(
        pl.kernel,
        out_type=jax.ShapeDtypeStruct((NC * n_pad, W), jnp.float32),
        mesh=mesh,
        compiler_params=pltpu.CompilerParams(use_tc_tiling_on_sc=False),
        scratch_types=[
            pltpu.VMEM((4, CH), jnp.int32),
            pltpu.VMEM((4, CH), jnp.int32),
            pltpu.VMEM((3 * CH, W), jnp.float32),
            pltpu.VMEM_SHARED((n_pad, W), jnp.float32),
            pltpu.SemaphoreType.DMA,
            pltpu.SemaphoreType.DMA,
            pltpu.SemaphoreType.DMA,
            pltpu.SemaphoreType.DMA,
        ],
    )
    def spmm(h_hbm, col_hbm, row_hbm, out_hbm, colv, rowv, buf, acc,
             isem, gsem0, gsem1, ssem):
        c = lax.axis_index("c")
        s = lax.axis_index("s")
        myk = jnp.where(c == 0, k0, k1)

        zoffs = list(range(0, W - 15, 16))
        if W % 16:
            zoffs.append(W - 16)

        def zrow(i, carry):
            for g in zoffs:
                buf[i, pl.ds(g, 16)] = jnp.zeros((16,), jnp.float32)
            return carry
        lax.fori_loop(0, CH, zrow, 0)

        r0 = s * rpt

        def zcopy(j, carry):
            pltpu.sync_copy(buf.at[pl.ds(0, CH)],
                            acc.at[pl.ds(r0 + j * CH, CH)])
            return carry
        lax.fori_loop(0, nfull, zcopy, 0)
        if rem:
            pltpu.sync_copy(buf.at[pl.ds(0, rem)],
                            acc.at[pl.ds(r0 + nfull * CH, rem)])

        cbase = jnp.where(c == 0, s * k0, NS * k0 + s * k1)

        def start_idx(j):
            sl = lax.rem(j, 4)
            pltpu.async_copy(col_hbm.at[cbase + j], colv.at[sl], isem)
            pltpu.async_copy(row_hbm.at[cbase + j], rowv.at[sl], isem)

        def sync_idx(j):
            sl = lax.rem(j, 4)
            pltpu.sync_copy(col_hbm.at[cbase + j], colv.at[sl])
            pltpu.sync_copy(row_hbm.at[cbase + j], rowv.at[sl])

        def wait_idx():
            for _ in range(2):
                pltpu.make_async_copy(col_hbm.at[0], colv.at[0], isem).wait()

        def bufsl(j):
            return buf.at[pl.ds(lax.rem(j, 3) * CH, CH)]

        def start_gather(j, sem):
            pltpu.async_copy(h_hbm.at[colv.at[lax.rem(j, 4)]], bufsl(j),
                             sem)

        def wait_gather(j, sem):
            pltpu.make_async_copy(
                h_hbm.at[colv.at[lax.rem(j, 4)]], bufsl(j), sem).wait()

        def wait_scatter(j):
            pltpu.make_async_copy(
                bufsl(j), acc.at[rowv.at[lax.rem(j, 4)]], ssem).wait()

        def even_odd(j, fn):
            @pl.when(lax.rem(j, 2) == 0)
            def _():
                fn(j, gsem0)

            @pl.when(lax.rem(j, 2) == 1)
            def _():
                fn(j, gsem1)

        sync_idx(0)
        sync_idx(1)
        start_idx(2)
        start_gather(0, gsem0)
        start_gather(1, gsem1)
        plsc.subcore_barrier()

        def echunk(j, carry):
            @pl.when(j >= 1)
            def _():
                wait_scatter(j - 1)

            even_odd(j, wait_gather)

            @pl.when(j + 2 < myk)
            def _():
                wait_idx()
                even_odd(j + 2, start_gather)

            @pl.when(j + 3 < myk)
            def _():
                start_idx(j + 3)

            pltpu.async_copy(bufsl(j), acc.at[rowv.at[lax.rem(j, 4)]],
                             ssem, add=True)
            return carry
        lax.fori_loop(0, myk, echunk, 0)
        wait_scatter(myk - 1)
        plsc.subcore_barrier()

        pltpu.sync_copy(acc.at[pl.ds(r0, rpt)],
                        out_hbm.at[pl.ds(c * n_pad + r0, rpt)])

    return spmm


def kernel(features, rel_emb, adj_index, none_relation):
    del rel_emb
    n, d = features.shape
    assert d == D and n % NS == 0
    e = adj_index.shape[0]
    grp = NS * CH
    e_pad = ((e + grp - 1) // grp) * grp
    n_pad = ((n + NS * 8 - 1) // (NS * 8)) * (NS * 8)
    k0 = round(e_pad // grp * SPLIT0)

    adj = adj_index.astype(jnp.int32)
    colp = jnp.concatenate(
        [adj[:, 1], jnp.full((e_pad - e,), n, jnp.int32)]).reshape(-1, CH)
    rowp = jnp.concatenate(
        [adj[:, 0], jnp.zeros((e_pad - e,), jnp.int32)]).reshape(-1, CH)
    nr2 = none_relation.reshape(1, D).astype(jnp.float32)
    n_h = n + 8

    spmm = _make_spmm(n_pad, e_pad, k0)

    def full_spec(a):
        return lambda b: pl.BlockSpec(a.shape, lambda i: (0, 0))

    def row_spec(width):
        return lambda b: pl.BlockSpec((b, width), lambda i: (i, 0))

    def part_spec(core):
        return lambda b: pl.BlockSpec((1, b, W), lambda i: (core, i, 0))

    prep = _make_prep_body(n, _row_block(n_h))
    finish = _make_finish_body(n, _row_block(n_h))

    f0, h0 = _dense_call(
        prep, n, n_h, [(features, row_spec(D)), (nr2, full_spec(nr2))])
    p = spmm(h0, colp, rowp).reshape(NC, n_pad, W)
    f1, h1 = _dense_call(
        finish, n, n_h,
        [(p, part_spec(0)), (p, part_spec(1)), (nr2, full_spec(nr2))])
    p2 = spmm(h1, colp, rowp).reshape(NC, n_pad, W)
    f2, _ = _dense_call(
        finish, n, n_h,
        [(p2, part_spec(0)), (p2, part_spec(1)), (nr2, full_spec(nr2))])
    return jnp.concatenate([f0, f1, f2], axis=1)


# speedup vs baseline: 1.5950x; 1.0975x over previous
import functools

import jax
import jax.numpy as jnp
from jax import lax
from jax.experimental import pallas as pl
from jax.experimental.pallas import tpu as pltpu
from jax.experimental.pallas import tpu_sc as plsc

D = 128
W = 144
NC = 2
NS = 16
CH = 80
SPLIT0 = 0.52


def _row_block(n):
    for b in range(1200, 7, -8):
        if n % b == 0:
            return b
    return n


def _node_math(t, nr_raw, valid):
    nr = nr_raw / jnp.maximum(jnp.sqrt(jnp.sum(nr_raw * nr_raw)), 1e-12)
    p = jnp.sum(t * nr, axis=1, keepdims=True)
    q = jnp.sqrt(jnp.sum(t * t, axis=1, keepdims=True))
    w = jnp.exp(-p / jnp.maximum(q, 1e-12))
    lane = lax.broadcasted_iota(jnp.int32, (t.shape[0], W - D), 1)
    tail = jnp.where(lane == 0, w, jnp.zeros_like(w))
    h = jnp.concatenate([t * w, tail], axis=1)
    return jnp.where(valid, h, jnp.zeros_like(h))


def _valid_rows(n, b, rows):
    i = pl.program_id(0)
    rowid = i * b + lax.broadcasted_iota(jnp.int32, (rows, 1), 0)
    return rowid < n


def _make_prep_body(n, b):
    def body(x_ref, nr_ref, f_ref, h_ref):
        valid = _valid_rows(n, b, x_ref.shape[0])
        t = jnp.tanh(jnp.where(valid, x_ref[...], 0.0))
        f_ref[...] = t
        h_ref[...] = _node_math(t, nr_ref[...], valid)
    return body


def _make_finish_body(n, b):
    def body(p0_ref, p1_ref, nr_ref, f_ref, h_ref):
        valid = _valid_rows(n, b, p0_ref.shape[1])
        s = p0_ref[0] + p1_ref[0]
        t = jnp.tanh(s[:, :D] / jnp.maximum(s[:, D:D + 1], 1e-12))
        t = jnp.where(valid, t, 0.0)
        f_ref[...] = t
        h_ref[...] = _node_math(t, nr_ref[...], valid)
    return body


def _dense_call(body, n, n_h, ins):
    b = _row_block(n_h)
    grid = (n_h // b,)
    in_specs = []
    for a, spec in ins:
        in_specs.append(spec(b))
    out_specs = [pl.BlockSpec((b, D), lambda i: (i, 0)),
                 pl.BlockSpec((b, W), lambda i: (i, 0))]
    out_shape = [jax.ShapeDtypeStruct((n, D), jnp.float32),
                 jax.ShapeDtypeStruct((n_h, W), jnp.float32)]
    return pl.pallas_call(
        body, grid=grid, in_specs=in_specs, out_specs=out_specs,
        out_shape=out_shape)(*[a for a, _ in ins])


def _make_spmm(n_pad, e_pad, k0):
    tot = e_pad // (NS * CH)
    k1 = tot - k0
    rpt = n_pad // NS
    nfull = rpt // CH
    rem = rpt - nfull * CH
    mesh = plsc.VectorSubcoreMesh(core_axis_name="c", subcore_axis_name="s")

    @functools.partial(
        pl.kernel,
        out_type=jax.ShapeDtypeStruct((NC * n_pad, W), jnp.float32),
        mesh=mesh,
        compiler_params=pltpu.CompilerParams(use_tc_tiling_on_sc=False),
        scratch_types=[
            pltpu.VMEM((4, CH), jnp.int32),
            pltpu.VMEM((4, CH), jnp.int32),
            pltpu.VMEM((3 * CH, W), jnp.float32),
            pltpu.VMEM_SHARED((n_pad, W), jnp.float32),
            pltpu.SemaphoreType.DMA,
            pltpu.SemaphoreType.DMA,
            pltpu.SemaphoreType.DMA,
            pltpu.SemaphoreType.DMA,
        ],
    )
    def spmm(h_hbm, col_hbm, row_hbm, out_hbm, colv, rowv, buf, acc,
             isem, gsem0, gsem1, ssem):
        c = lax.axis_index("c")
        s = lax.axis_index("s")
        myk = jnp.where(c == 0, k0, k1)

        zoffs = list(range(0, W - 15, 16))
        if W % 16:
            zoffs.append(W - 16)

        def zrow(i, carry):
            for g in zoffs:
                buf[i, pl.ds(g, 16)] = jnp.zeros((16,), jnp.float32)
            return carry
        lax.fori_loop(0, CH, zrow, 0)

        r0 = s * rpt

        def zcopy(j, carry):
            pltpu.sync_copy(buf.at[pl.ds(0, CH)],
                            acc.at[pl.ds(r0 + j * CH, CH)])
            return carry
        lax.fori_loop(0, nfull, zcopy, 0)
        if rem:
            pltpu.sync_copy(buf.at[pl.ds(0, rem)],
                            acc.at[pl.ds(r0 + nfull * CH, rem)])

        cbase = jnp.where(c == 0, s * k0, NS * k0 + s * k1)

        def start_idx(j):
            sl = lax.rem(j, 4)
            pltpu.async_copy(col_hbm.at[cbase + j], colv.at[sl], isem)
            pltpu.async_copy(row_hbm.at[cbase + j], rowv.at[sl], isem)

        def sync_idx(j):
            sl = lax.rem(j, 4)
            pltpu.sync_copy(col_hbm.at[cbase + j], colv.at[sl])
            pltpu.sync_copy(row_hbm.at[cbase + j], rowv.at[sl])

        def wait_idx():
            for _ in range(2):
                pltpu.make_async_copy(col_hbm.at[0], colv.at[0], isem).wait()

        def bufsl(j):
            return buf.at[pl.ds(lax.rem(j, 3) * CH, CH)]

        def start_gather(j, sem):
            pltpu.async_copy(h_hbm.at[colv.at[lax.rem(j, 4)]], bufsl(j),
                             sem)

        def wait_gather(j, sem):
            pltpu.make_async_copy(
                h_hbm.at[colv.at[lax.rem(j, 4)]], bufsl(j), sem).wait()

        def wait_scatter(j):
            pltpu.make_async_copy(
                bufsl(j), acc.at[rowv.at[lax.rem(j, 4)]], ssem).wait()

        def even_odd(j, fn):
            @pl.when(lax.rem(j, 2) == 0)
            def _():
                fn(j, gsem0)

            @pl.when(lax.rem(j, 2) == 1)
            def _():
                fn(j, gsem1)

        sync_idx(0)
        sync_idx(1)
        start_idx(2)
        start_gather(0, gsem0)
        start_gather(1, gsem1)
        plsc.subcore_barrier()

        def echunk(j, carry):
            @pl.when(j >= 1)
            def _():
                wait_scatter(j - 1)

            even_odd(j, wait_gather)

            @pl.when(j + 2 < myk)
            def _():
                wait_idx()
                even_odd(j + 2, start_gather)

            @pl.when(j + 3 < myk)
            def _():
                start_idx(j + 3)

            pltpu.async_copy(bufsl(j), acc.at[rowv.at[lax.rem(j, 4)]],
                             ssem, add=True)
            return carry
        lax.fori_loop(0, myk, echunk, 0)
        wait_scatter(myk - 1)
        plsc.subcore_barrier()

        pltpu.sync_copy(acc.at[pl.ds(r0, rpt)],
                        out_hbm.at[pl.ds(c * n_pad + r0, rpt)])

    return spmm


def kernel(features, rel_emb, adj_index, none_relation):
    del rel_emb
    n, d = features.shape
    assert d == D and n % NS == 0
    e = adj_index.shape[0]
    grp = NS * CH
    e_pad = ((e + grp - 1) // grp) * grp
    n_pad = ((n + NS * 8 - 1) // (NS * 8)) * (NS * 8)
    k0 = round(e_pad // grp * SPLIT0)

    adj = adj_index.astype(jnp.int32)
    colp = jnp.concatenate(
        [adj[:, 1], jnp.full((e_pad - e,), n, jnp.int32)]).reshape(-1, CH)
    rowp = jnp.concatenate(
        [adj[:, 0], jnp.zeros((e_pad - e,), jnp.int32)]).reshape(-1, CH)
    nr2 = none_relation.reshape(1, D).astype(jnp.float32)
    n_h = n + 8

    spmm = _make_spmm(n_pad, e_pad, k0)

    def full_spec(a):
        return lambda b: pl.BlockSpec(a.shape, lambda i: (0, 0))

    def row_spec(width):
        return lambda b: pl.BlockSpec((b, width), lambda i: (i, 0))

    def part_spec(core):
        return lambda b: pl.BlockSpec((1, b, W), lambda i: (core, i, 0))

    prep = _make_prep_body(n, _row_block(n_h))
    finish = _make_finish_body(n, _row_block(n_h))

    f0, h0 = _dense_call(
        prep, n, n_h, [(features, row_spec(D)), (nr2, full_spec(nr2))])
    p = spmm(h0, colp, rowp).reshape(NC, n_pad, W)
    f1, h1 = _dense_call(
        finish, n, n_h,
        [(p, part_spec(0)), (p, part_spec(1)), (nr2, full_spec(nr2))])
    p2 = spmm(h1, colp, rowp).reshape(NC, n_pad, W)
    f2, _ = _dense_call(
        finish, n, n_h,
        [(p2, part_spec(0)), (p2, part_spec(1)), (nr2, full_spec(nr2))])
    return jnp.concatenate([f0, f1, f2], axis=1)


# speedup vs baseline: 1.5999x; 1.0031x over previous
import functools

import jax
import jax.numpy as jnp
from jax import lax
from jax.experimental import pallas as pl
from jax.experimental.pallas import tpu as pltpu
from jax.experimental.pallas import tpu_sc as plsc

D = 128
W = 144
NC = 2
NS = 16
CH = 80
SPLIT0 = 0.52


def _row_block(n):
    for b in range(1200, 7, -8):
        if n % b == 0:
            return b
    return n


def _node_math(t, nr_raw, valid):
    nr = nr_raw / jnp.maximum(jnp.sqrt(jnp.sum(nr_raw * nr_raw)), 1e-12)
    p = jnp.sum(t * nr, axis=1, keepdims=True)
    q = jnp.sqrt(jnp.sum(t * t, axis=1, keepdims=True))
    w = jnp.exp(-p / jnp.maximum(q, 1e-12))
    lane = lax.broadcasted_iota(jnp.int32, (t.shape[0], W - D), 1)
    tail = jnp.where(lane == 0, w, jnp.zeros_like(w))
    h = jnp.concatenate([t * w, tail], axis=1)
    return jnp.where(valid, h, jnp.zeros_like(h))


def _valid_rows(n, b, rows):
    i = pl.program_id(0)
    rowid = i * b + lax.broadcasted_iota(jnp.int32, (rows, 1), 0)
    return rowid < n


def _make_prep_body(n, b):
    def body(x_ref, nr_ref, ob_ref, f_ref, h_ref):
        del ob_ref
        valid = _valid_rows(n, b, x_ref.shape[0])
        t = jnp.tanh(jnp.where(valid, x_ref[...], 0.0))
        f_ref[...] = t
        h_ref[...] = _node_math(t, nr_ref[...], valid)
    return body


def _make_finish_body(n, b):
    def body(p0_ref, p1_ref, nr_ref, ob_ref, f_ref, h_ref):
        del ob_ref
        valid = _valid_rows(n, b, p0_ref.shape[1])
        s = p0_ref[0] + p1_ref[0]
        t = jnp.tanh(s[:, :D] / jnp.maximum(s[:, D:D + 1], 1e-12))
        t = jnp.where(valid, t, 0.0)
        f_ref[...] = t
        h_ref[...] = _node_math(t, nr_ref[...], valid)
    return body


def _dense_call(body, n, n_h, ins, layer, obuf):
    b = _row_block(n_h)
    grid = (n_h // b,)
    in_specs = [spec(b) for _, spec in ins]
    in_specs.append(pl.BlockSpec((b, D), lambda i: (i, layer)))
    out_specs = [pl.BlockSpec((b, D), lambda i: (i, layer)),
                 pl.BlockSpec((b, W), lambda i: (i, 0))]
    out_shape = [jax.ShapeDtypeStruct(obuf.shape, jnp.float32),
                 jax.ShapeDtypeStruct((n_h, W), jnp.float32)]
    nin = len(ins)
    return pl.pallas_call(
        body, grid=grid, in_specs=in_specs, out_specs=out_specs,
        out_shape=out_shape,
        input_output_aliases={nin: 0})(*([a for a, _ in ins] + [obuf]))


def _make_spmm(n_pad, e_pad, k0):
    tot = e_pad // (NS * CH)
    k1 = tot - k0
    rpt = n_pad // NS
    nfull = rpt // CH
    rem = rpt - nfull * CH
    mesh = plsc.VectorSubcoreMesh(core_axis_name="c", subcore_axis_name="s")

    @functools.partial(
        pl.kernel,
        out_type=jax.ShapeDtypeStruct((NC * n_pad, W), jnp.float32),
        mesh=mesh,
        compiler_params=pltpu.CompilerParams(use_tc_tiling_on_sc=False),
        scratch_types=[
            pltpu.VMEM((4, CH), jnp.int32),
            pltpu.VMEM((4, CH), jnp.int32),
            pltpu.VMEM((3 * CH, W), jnp.float32),
            pltpu.VMEM_SHARED((n_pad, W), jnp.float32),
            pltpu.SemaphoreType.DMA,
            pltpu.SemaphoreType.DMA,
            pltpu.SemaphoreType.DMA,
            pltpu.SemaphoreType.DMA,
        ],
    )
    def spmm(h_hbm, col_hbm, row_hbm, out_hbm, colv, rowv, buf, acc,
             isem, gsem0, gsem1, ssem):
        c = lax.axis_index("c")
        s = lax.axis_index("s")
        myk = jnp.where(c == 0, k0, k1)

        zoffs = list(range(0, W - 15, 16))
        if W % 16:
            zoffs.append(W - 16)

        def zrow(i, carry):
            for g in zoffs:
                buf[i, pl.ds(g, 16)] = jnp.zeros((16,), jnp.float32)
            return carry
        lax.fori_loop(0, CH, zrow, 0)

        r0 = s * rpt

        def zcopy(j, carry):
            pltpu.sync_copy(buf.at[pl.ds(0, CH)],
                            acc.at[pl.ds(r0 + j * CH, CH)])
            return carry
        lax.fori_loop(0, nfull, zcopy, 0)
        if rem:
            pltpu.sync_copy(buf.at[pl.ds(0, rem)],
                            acc.at[pl.ds(r0 + nfull * CH, rem)])

        cbase = jnp.where(c == 0, s * k0, NS * k0 + s * k1)

        def start_idx(j):
            sl = lax.rem(j, 4)
            pltpu.async_copy(col_hbm.at[cbase + j], colv.at[sl], isem)
            pltpu.async_copy(row_hbm.at[cbase + j], rowv.at[sl], isem)

        def sync_idx(j):
            sl = lax.rem(j, 4)
            pltpu.sync_copy(col_hbm.at[cbase + j], colv.at[sl])
            pltpu.sync_copy(row_hbm.at[cbase + j], rowv.at[sl])

        def wait_idx():
            for _ in range(2):
                pltpu.make_async_copy(col_hbm.at[0], colv.at[0], isem).wait()

        def bufsl(j):
            return buf.at[pl.ds(lax.rem(j, 3) * CH, CH)]

        def start_gather(j, sem):
            pltpu.async_copy(h_hbm.at[colv.at[lax.rem(j, 4)]], bufsl(j),
                             sem)

        def wait_gather(j, sem):
            pltpu.make_async_copy(
                h_hbm.at[colv.at[lax.rem(j, 4)]], bufsl(j), sem).wait()

        def wait_scatter(j):
            pltpu.make_async_copy(
                bufsl(j), acc.at[rowv.at[lax.rem(j, 4)]], ssem).wait()

        def even_odd(j, fn):
            @pl.when(lax.rem(j, 2) == 0)
            def _():
                fn(j, gsem0)

            @pl.when(lax.rem(j, 2) == 1)
            def _():
                fn(j, gsem1)

        sync_idx(0)
        sync_idx(1)
        start_idx(2)
        start_gather(0, gsem0)
        start_gather(1, gsem1)
        plsc.subcore_barrier()

        def echunk(j, carry):
            @pl.when(j >= 1)
            def _():
                wait_scatter(j - 1)

            even_odd(j, wait_gather)

            @pl.when(j + 2 < myk)
            def _():
                wait_idx()
                even_odd(j + 2, start_gather)

            @pl.when(j + 3 < myk)
            def _():
                start_idx(j + 3)

            pltpu.async_copy(bufsl(j), acc.at[rowv.at[lax.rem(j, 4)]],
                             ssem, add=True)
            return carry
        lax.fori_loop(0, myk, echunk, 0)
        wait_scatter(myk - 1)
        plsc.subcore_barrier()

        pltpu.sync_copy(acc.at[pl.ds(r0, rpt)],
                        out_hbm.at[pl.ds(c * n_pad + r0, rpt)])

    return spmm


def kernel(features, rel_emb, adj_index, none_relation):
    del rel_emb
    n, d = features.shape
    assert d == D and n % NS == 0
    e = adj_index.shape[0]
    grp = NS * CH
    e_pad = ((e + grp - 1) // grp) * grp
    n_pad = ((n + NS * 8 - 1) // (NS * 8)) * (NS * 8)
    k0 = round(e_pad // grp * SPLIT0)

    adj = adj_index.astype(jnp.int32)
    colp = jnp.concatenate(
        [adj[:, 1], jnp.full((e_pad - e,), n, jnp.int32)]).reshape(-1, CH)
    rowp = jnp.concatenate(
        [adj[:, 0], jnp.zeros((e_pad - e,), jnp.int32)]).reshape(-1, CH)
    nr2 = none_relation.reshape(1, D).astype(jnp.float32)
    n_h = n + 8

    spmm = _make_spmm(n_pad, e_pad, k0)

    def full_spec(a):
        return lambda b: pl.BlockSpec(a.shape, lambda i: (0, 0))

    def row_spec(width):
        return lambda b: pl.BlockSpec((b, width), lambda i: (i, 0))

    def part_spec(core):
        return lambda b: pl.BlockSpec((1, b, W), lambda i: (core, i, 0))

    prep = _make_prep_body(n, _row_block(n_h))
    finish = _make_finish_body(n, _row_block(n_h))

    ob = jnp.zeros((n, 3 * D), jnp.float32)
    ob, h0 = _dense_call(
        prep, n, n_h, [(features, row_spec(D)), (nr2, full_spec(nr2))],
        0, ob)
    p = spmm(h0, colp, rowp).reshape(NC, n_pad, W)
    ob, h1 = _dense_call(
        finish, n, n_h,
        [(p, part_spec(0)), (p, part_spec(1)), (nr2, full_spec(nr2))],
        1, ob)
    p2 = spmm(h1, colp, rowp).reshape(NC, n_pad, W)
    ob, _ = _dense_call(
        finish, n, n_h,
        [(p2, part_spec(0)), (p2, part_spec(1)), (nr2, full_spec(nr2))],
        2, ob)
    return ob
